# baseline (speedup 1.0000x reference)
import jax
import jax.numpy as jnp
from jax.experimental import pallas as pl
from jax.experimental.pallas import tpu as pltpu

NQ = 8
K = 1024
D = 256
B = 16
S = 1536


def _tree_sum_sublanes(x):
    sz = x.shape[0]
    while sz > 1:
        sz //= 2
        x = x[:sz] + x[sz:]
    return x


def _tree_sum_lanes(x):
    sz = x.shape[-1]
    while sz > 1:
        sz //= 2
        x = x[..., :sz] + x[..., sz:]
    return x


def _rvq_body(z_ref, cb_ref, q_ref, idx_ref, loss_ref, perp_ref,
              counts_ref, loss_acc_ref, cn_ref, mid_ref, low_ref):
    b = pl.program_id(0)
    nsteps = pl.num_programs(0)
    first = b == 0
    last = b == nsteps - 1

    @pl.when(first)
    def _init():
        counts_ref[...] = jnp.zeros_like(counts_ref)
        loss_acc_ref[...] = jnp.zeros_like(loss_acc_ref)
        cb = cb_ref[...]
        cn_ref[...] = _tree_sum_lanes(cb * cb)[..., 0]
        hi = cb.astype(jnp.bfloat16).astype(jnp.float32)
        mid_f = (cb - hi).astype(jnp.bfloat16).astype(jnp.float32)
        mid_ref[...] = mid_f.astype(jnp.bfloat16)
        low_ref[...] = (cb - hi - mid_f).astype(jnp.bfloat16)

    r = z_ref[0]
    quant = jnp.zeros_like(r)
    iota = jax.lax.broadcasted_iota(jnp.int32, (K, S), 0)
    for q in range(NQ):
        c = cb_ref[q]
        xc = jax.lax.dot_general(
            c, r, (((1,), (0,)), ((), ())),
            preferred_element_type=jnp.float32,
            precision=jax.lax.Precision.DEFAULT)
        rn = _tree_sum_sublanes(r * r)
        d = (rn - 2.0 * xc) + cn_ref[q][:, None]
        m = jnp.min(d, axis=0, keepdims=True)
        idx = jnp.min(jnp.where(d == m, iota, K), axis=0)
        onehot = jnp.where(iota == idx[None, :], 1.0, 0.0)
        onehot_b = onehot.astype(jnp.bfloat16)
        sel = jax.lax.dot_general(
            c, onehot, (((0,), (0,)), ((), ())),
            preferred_element_type=jnp.float32,
            precision=jax.lax.Precision.DEFAULT)
        for plane in (mid_ref[q], low_ref[q]):
            part = jax.lax.dot_general(
                plane, onehot_b, (((0,), (0,)), ((), ())),
                preferred_element_type=jnp.float32,
                precision=jax.lax.Precision.DEFAULT)
            sel = sel + part
        quant = quant + sel
        r = r - sel
        idx_ref[0, q, :] = idx
        counts_ref[q, :] += jnp.sum(onehot, axis=1)

    q_ref[0] = quant
    loss_acc_ref[...] = loss_acc_ref[...] + jnp.sum(r * r)

    @pl.when(last)
    def _epilogue():
        counts = counts_ref[...]
        total = jnp.clip(jnp.sum(counts, axis=1, keepdims=True), 1.0, None)
        probs = counts / total
        ent = -jnp.sum(probs * jnp.log(probs + 1e-10), axis=1, keepdims=True)
        perp_ref[...] = jnp.mean(jnp.exp(ent)).reshape(1, 1)
        loss_ref[...] = loss_acc_ref[...] / (B * S * D)


@jax.jit
def kernel(z, codebooks):
    quant, idx, loss, perp = pl.pallas_call(
        _rvq_body,
        grid=(B,),
        in_specs=[
            pl.BlockSpec((1, D, S), lambda b: (b, 0, 0)),
            pl.BlockSpec((NQ, K, D), lambda b: (0, 0, 0)),
        ],
        out_specs=[
            pl.BlockSpec((1, D, S), lambda b: (b, 0, 0)),
            pl.BlockSpec((1, NQ, S), lambda b: (b, 0, 0)),
            pl.BlockSpec((1, 1), lambda b: (0, 0)),
            pl.BlockSpec((1, 1), lambda b: (0, 0)),
        ],
        out_shape=[
            jax.ShapeDtypeStruct((B, D, S), jnp.float32),
            jax.ShapeDtypeStruct((B, NQ, S), jnp.int32),
            jax.ShapeDtypeStruct((1, 1), jnp.float32),
            jax.ShapeDtypeStruct((1, 1), jnp.float32),
        ],
        scratch_shapes=[
            pltpu.VMEM((NQ, K), jnp.float32),
            pltpu.VMEM((1, 1), jnp.float32),
            pltpu.VMEM((NQ, K), jnp.float32),
            pltpu.VMEM((NQ, K, D), jnp.bfloat16),
            pltpu.VMEM((NQ, K, D), jnp.bfloat16),
        ],
    )(z, codebooks)
    return (quant,
            jnp.transpose(idx, (0, 2, 1)),
            loss[0, 0],
            perp[0, 0])


# speedup vs baseline: 1.0377x; 1.0377x over previous
import jax
import jax.numpy as jnp
from jax.experimental import pallas as pl
from jax.experimental.pallas import tpu as pltpu

NQ = 8
K = 1024
D = 256
B = 16
S = 1536
H = S // 2


def _tree_sum_sublanes(x):
    sz = x.shape[0]
    while sz > 1:
        sz //= 2
        x = x[:sz] + x[sz:]
    return x


def _tree_sum_lanes(x):
    sz = x.shape[-1]
    while sz > 1:
        sz //= 2
        x = x[..., :sz] + x[..., sz:]
    return x


def _rvq_body(z_ref, cb_ref, q_ref, idx_ref, loss_ref, perp_ref,
              counts_ref, loss_acc_ref, cn_ref, mid_ref, low_ref):
    b = pl.program_id(0)
    nsteps = pl.num_programs(0)
    first = b == 0
    last = b == nsteps - 1

    @pl.when(first)
    def _init():
        counts_ref[...] = jnp.zeros_like(counts_ref)
        loss_acc_ref[...] = jnp.zeros_like(loss_acc_ref)
        cb = cb_ref[...]
        cn_ref[...] = _tree_sum_lanes(cb * cb)[..., 0]
        hi = cb.astype(jnp.bfloat16).astype(jnp.float32)
        mid_f = (cb - hi).astype(jnp.bfloat16).astype(jnp.float32)
        mid_ref[...] = mid_f.astype(jnp.bfloat16)
        low_ref[...] = (cb - hi - mid_f).astype(jnp.bfloat16)

    iota = jax.lax.broadcasted_iota(jnp.int32, (K, H), 0)

    def chain_scores(c, cn, r):
        xc = jax.lax.dot_general(
            c, r, (((1,), (0,)), ((), ())),
            preferred_element_type=jnp.float32,
            precision=jax.lax.Precision.DEFAULT)
        rn = _tree_sum_sublanes(r * r)
        return (rn - 2.0 * xc) + cn[:, None]

    def chain_select(d):
        m = jnp.min(d, axis=0, keepdims=True)
        idx = jnp.min(jnp.where(d == m, iota, K), axis=0)
        onehot = jnp.where(iota == idx[None, :], 1.0, 0.0)
        return idx, onehot

    def chain_gather(c, q_idx, onehot):
        onehot_b = onehot.astype(jnp.bfloat16)
        sel = jax.lax.dot_general(
            c, onehot, (((0,), (0,)), ((), ())),
            preferred_element_type=jnp.float32,
            precision=jax.lax.Precision.DEFAULT)
        for plane in (mid_ref[q_idx], low_ref[q_idx]):
            part = jax.lax.dot_general(
                plane, onehot_b, (((0,), (0,)), ((), ())),
                preferred_element_type=jnp.float32,
                precision=jax.lax.Precision.DEFAULT)
            sel = sel + part
        return sel

    r_a = z_ref[0, :, :H]
    r_b = z_ref[0, :, H:]
    quant_a = jnp.zeros_like(r_a)
    quant_b = jnp.zeros_like(r_b)
    for q in range(NQ):
        c = cb_ref[q]
        cn = cn_ref[q]
        d_a = chain_scores(c, cn, r_a)
        d_b = chain_scores(c, cn, r_b)
        idx_a, onehot_a = chain_select(d_a)
        sel_a = chain_gather(c, q, onehot_a)
        idx_b, onehot_b = chain_select(d_b)
        sel_b = chain_gather(c, q, onehot_b)
        quant_a = quant_a + sel_a
        r_a = r_a - sel_a
        quant_b = quant_b + sel_b
        r_b = r_b - sel_b
        idx_ref[0, q, :H] = idx_a
        idx_ref[0, q, H:] = idx_b
        counts_ref[q, :] += (jnp.sum(onehot_a, axis=1)
                             + jnp.sum(onehot_b, axis=1))

    q_ref[0, :, :H] = quant_a
    q_ref[0, :, H:] = quant_b
    loss_acc_ref[...] = (loss_acc_ref[...] + jnp.sum(r_a * r_a)
                         + jnp.sum(r_b * r_b))

    @pl.when(last)
    def _epilogue():
        counts = counts_ref[...]
        total = jnp.clip(jnp.sum(counts, axis=1, keepdims=True), 1.0, None)
        probs = counts / total
        ent = -jnp.sum(probs * jnp.log(probs + 1e-10), axis=1, keepdims=True)
        perp_ref[...] = jnp.mean(jnp.exp(ent)).reshape(1, 1)
        loss_ref[...] = loss_acc_ref[...] / (B * S * D)


@jax.jit
def kernel(z, codebooks):
    quant, idx, loss, perp = pl.pallas_call(
        _rvq_body,
        grid=(B,),
        in_specs=[
            pl.BlockSpec((1, D, S), lambda b: (b, 0, 0)),
            pl.BlockSpec((NQ, K, D), lambda b: (0, 0, 0)),
        ],
        out_specs=[
            pl.BlockSpec((1, D, S), lambda b: (b, 0, 0)),
            pl.BlockSpec((1, NQ, S), lambda b: (b, 0, 0)),
            pl.BlockSpec((1, 1), lambda b: (0, 0)),
            pl.BlockSpec((1, 1), lambda b: (0, 0)),
        ],
        out_shape=[
            jax.ShapeDtypeStruct((B, D, S), jnp.float32),
            jax.ShapeDtypeStruct((B, NQ, S), jnp.int32),
            jax.ShapeDtypeStruct((1, 1), jnp.float32),
            jax.ShapeDtypeStruct((1, 1), jnp.float32),
        ],
        scratch_shapes=[
            pltpu.VMEM((NQ, K), jnp.float32),
            pltpu.VMEM((1, 1), jnp.float32),
            pltpu.VMEM((NQ, K), jnp.float32),
            pltpu.VMEM((NQ, K, D), jnp.bfloat16),
            pltpu.VMEM((NQ, K, D), jnp.bfloat16),
        ],
    )(z, codebooks)
    return (quant,
            jnp.transpose(idx, (0, 2, 1)),
            loss[0, 0],
            perp[0, 0])


# speedup vs baseline: 1.0507x; 1.0126x over previous
import jax
import jax.numpy as jnp
from jax.experimental import pallas as pl
from jax.experimental.pallas import tpu as pltpu

NQ = 8
K = 1024
D = 256
B = 16
S = 1536
H = S // 2


def _tree_sum_sublanes(x):
    sz = x.shape[0]
    while sz > 1:
        sz //= 2
        x = x[:sz] + x[sz:]
    return x


def _tree_sum_lanes(x):
    sz = x.shape[-1]
    while sz > 1:
        sz //= 2
        x = x[..., :sz] + x[..., sz:]
    return x


def _rvq_body(z_ref, cb_ref, q_ref, idx_ref, loss_ref, perp_ref,
              counts_ref, loss_acc_ref, cn_ref, c2_ref, mid_ref, low_ref):
    b = pl.program_id(0)
    nsteps = pl.num_programs(0)
    first = b == 0
    last = b == nsteps - 1

    @pl.when(first)
    def _init():
        counts_ref[...] = jnp.zeros_like(counts_ref)
        loss_acc_ref[...] = jnp.zeros_like(loss_acc_ref)
        cb = cb_ref[...]
        cn_ref[...] = _tree_sum_lanes(cb * cb)[..., 0]
        c2_ref[...] = (2.0 * cb).astype(jnp.bfloat16)
        hi = cb.astype(jnp.bfloat16).astype(jnp.float32)
        mid_f = (cb - hi).astype(jnp.bfloat16).astype(jnp.float32)
        mid_ref[...] = mid_f.astype(jnp.bfloat16)
        low_ref[...] = (cb - hi - mid_f).astype(jnp.bfloat16)

    iota = jax.lax.broadcasted_iota(jnp.int32, (K, H), 0)

    def chain_scores(q, r):
        xc2 = jax.lax.dot_general(
            c2_ref[q], r.astype(jnp.bfloat16), (((1,), (0,)), ((), ())),
            preferred_element_type=jnp.float32)
        rn = _tree_sum_sublanes(r * r)
        return (rn - xc2) + cn_ref[q][:, None]

    def chain_select(d):
        m = jnp.min(d, axis=0, keepdims=True)
        idx = jnp.min(jnp.where(d == m, iota, K), axis=0)
        onehot = jnp.where(iota == idx[None, :], 1.0, 0.0)
        return idx, onehot

    def chain_gather(c, q_idx, onehot):
        onehot_b = onehot.astype(jnp.bfloat16)
        sel = jax.lax.dot_general(
            c, onehot, (((0,), (0,)), ((), ())),
            preferred_element_type=jnp.float32,
            precision=jax.lax.Precision.DEFAULT)
        for plane in (mid_ref[q_idx], low_ref[q_idx]):
            part = jax.lax.dot_general(
                plane, onehot_b, (((0,), (0,)), ((), ())),
                preferred_element_type=jnp.float32)
            sel = sel + part
        return sel

    r_a = z_ref[0, :, :H]
    r_b = z_ref[0, :, H:]
    quant_a = jnp.zeros_like(r_a)
    quant_b = jnp.zeros_like(r_b)
    for q in range(NQ):
        c = cb_ref[q]
        d_a = chain_scores(q, r_a)
        d_b = chain_scores(q, r_b)
        idx_a, onehot_a = chain_select(d_a)
        sel_a = chain_gather(c, q, onehot_a)
        idx_b, onehot_b = chain_select(d_b)
        sel_b = chain_gather(c, q, onehot_b)
        quant_a = quant_a + sel_a
        r_a = r_a - sel_a
        quant_b = quant_b + sel_b
        r_b = r_b - sel_b
        idx_ref[0, q, :H] = idx_a
        idx_ref[0, q, H:] = idx_b
        counts_ref[q, :] += (jnp.sum(onehot_a, axis=1)
                             + jnp.sum(onehot_b, axis=1))

    q_ref[0, :, :H] = quant_a
    q_ref[0, :, H:] = quant_b
    loss_acc_ref[...] = (loss_acc_ref[...] + jnp.sum(r_a * r_a)
                         + jnp.sum(r_b * r_b))

    @pl.when(last)
    def _epilogue():
        counts = counts_ref[...]
        total = jnp.clip(jnp.sum(counts, axis=1, keepdims=True), 1.0, None)
        probs = counts / total
        ent = -jnp.sum(probs * jnp.log(probs + 1e-10), axis=1, keepdims=True)
        perp_ref[...] = jnp.mean(jnp.exp(ent)).reshape(1, 1)
        loss_ref[...] = loss_acc_ref[...] / (B * S * D)


@jax.jit
def kernel(z, codebooks):
    quant, idx, loss, perp = pl.pallas_call(
        _rvq_body,
        grid=(B,),
        in_specs=[
            pl.BlockSpec((1, D, S), lambda b: (b, 0, 0)),
            pl.BlockSpec((NQ, K, D), lambda b: (0, 0, 0)),
        ],
        out_specs=[
            pl.BlockSpec((1, D, S), lambda b: (b, 0, 0)),
            pl.BlockSpec((1, NQ, S), lambda b: (b, 0, 0)),
            pl.BlockSpec((1, 1), lambda b: (0, 0)),
            pl.BlockSpec((1, 1), lambda b: (0, 0)),
        ],
        out_shape=[
            jax.ShapeDtypeStruct((B, D, S), jnp.float32),
            jax.ShapeDtypeStruct((B, NQ, S), jnp.int32),
            jax.ShapeDtypeStruct((1, 1), jnp.float32),
            jax.ShapeDtypeStruct((1, 1), jnp.float32),
        ],
        scratch_shapes=[
            pltpu.VMEM((NQ, K), jnp.float32),
            pltpu.VMEM((1, 1), jnp.float32),
            pltpu.VMEM((NQ, K), jnp.float32),
            pltpu.VMEM((NQ, K, D), jnp.bfloat16),
            pltpu.VMEM((NQ, K, D), jnp.bfloat16),
            pltpu.VMEM((NQ, K, D), jnp.bfloat16),
        ],
    )(z, codebooks)
    return (quant,
            jnp.transpose(idx, (0, 2, 1)),
            loss[0, 0],
            perp[0, 0])


# speedup vs baseline: 1.0832x; 1.0310x over previous
import jax
import jax.numpy as jnp
from jax.experimental import pallas as pl
from jax.experimental.pallas import tpu as pltpu

NQ = 8
K = 1024
D = 256
B = 16
S = 1536
H = S // 2


def _tree_sum_sublanes(x):
    sz = x.shape[0]
    while sz > 1:
        sz //= 2
        x = x[:sz] + x[sz:]
    return x


def _tree_sum_lanes(x):
    sz = x.shape[-1]
    while sz > 1:
        sz //= 2
        x = x[..., :sz] + x[..., sz:]
    return x


def _rvq_body(z_ref, cb_ref, q_ref, idx_ref, loss_ref, perp_ref,
              counts_ref, loss_acc_ref, cn_ref, c2_ref, mid_ref, low_ref):
    b = pl.program_id(0)
    nsteps = pl.num_programs(0)
    first = b == 0
    last = b == nsteps - 1

    @pl.when(first)
    def _init():
        counts_ref[...] = jnp.zeros_like(counts_ref)
        loss_acc_ref[...] = jnp.zeros_like(loss_acc_ref)
        cb = cb_ref[...]
        cn_ref[...] = _tree_sum_lanes(cb * cb)[..., 0]
        c2_ref[...] = (2.0 * cb).astype(jnp.bfloat16)
        hi = cb.astype(jnp.bfloat16).astype(jnp.float32)
        mid_f = (cb - hi).astype(jnp.bfloat16).astype(jnp.float32)
        mid_ref[...] = mid_f.astype(jnp.bfloat16)
        low_ref[...] = (cb - hi - mid_f).astype(jnp.bfloat16)

    iota = jax.lax.broadcasted_iota(jnp.int32, (K, H), 0)

    def chain_scores(q, r):
        xc2 = jax.lax.dot_general(
            c2_ref[q], r.astype(jnp.bfloat16), (((1,), (0,)), ((), ())),
            preferred_element_type=jnp.float32)
        rn = _tree_sum_sublanes(r * r)
        return (rn - xc2) + cn_ref[q][:, None]

    def chain_select(d):
        m = jnp.min(d, axis=0, keepdims=True)
        idx = jnp.min(jnp.where(d == m, iota, K), axis=0)
        onehot = (iota == idx[None, :]).astype(jnp.bfloat16)
        return idx, onehot

    def chain_gather(q_idx, onehot):
        sel2 = jax.lax.dot_general(
            c2_ref[q_idx], onehot, (((0,), (0,)), ((), ())),
            preferred_element_type=jnp.float32)
        sel = 0.5 * sel2
        for plane in (mid_ref[q_idx], low_ref[q_idx]):
            part = jax.lax.dot_general(
                plane, onehot, (((0,), (0,)), ((), ())),
                preferred_element_type=jnp.float32)
            sel = sel + part
        return sel

    r_a = z_ref[0, :, :H]
    r_b = z_ref[0, :, H:]
    quant_a = jnp.zeros_like(r_a)
    quant_b = jnp.zeros_like(r_b)
    for q in range(NQ):
        d_a = chain_scores(q, r_a)
        d_b = chain_scores(q, r_b)
        idx_a, onehot_a = chain_select(d_a)
        sel_a = chain_gather(q, onehot_a)
        idx_b, onehot_b = chain_select(d_b)
        sel_b = chain_gather(q, onehot_b)
        quant_a = quant_a + sel_a
        r_a = r_a - sel_a
        quant_b = quant_b + sel_b
        r_b = r_b - sel_b
        idx_ref[0, q, :H] = idx_a
        idx_ref[0, q, H:] = idx_b
        counts_ref[q, :] += (jnp.sum(onehot_a, axis=1, dtype=jnp.float32)
                             + jnp.sum(onehot_b, axis=1, dtype=jnp.float32))

    q_ref[0, :, :H] = quant_a
    q_ref[0, :, H:] = quant_b
    loss_acc_ref[...] = (loss_acc_ref[...] + jnp.sum(r_a * r_a)
                         + jnp.sum(r_b * r_b))

    @pl.when(last)
    def _epilogue():
        counts = counts_ref[...]
        total = jnp.clip(jnp.sum(counts, axis=1, keepdims=True), 1.0, None)
        probs = counts / total
        ent = -jnp.sum(probs * jnp.log(probs + 1e-10), axis=1, keepdims=True)
        perp_ref[...] = jnp.mean(jnp.exp(ent)).reshape(1, 1)
        loss_ref[...] = loss_acc_ref[...] / (B * S * D)


@jax.jit
def kernel(z, codebooks):
    quant, idx, loss, perp = pl.pallas_call(
        _rvq_body,
        grid=(B,),
        in_specs=[
            pl.BlockSpec((1, D, S), lambda b: (b, 0, 0)),
            pl.BlockSpec((NQ, K, D), lambda b: (0, 0, 0)),
        ],
        out_specs=[
            pl.BlockSpec((1, D, S), lambda b: (b, 0, 0)),
            pl.BlockSpec((1, NQ, S), lambda b: (b, 0, 0)),
            pl.BlockSpec((1, 1), lambda b: (0, 0)),
            pl.BlockSpec((1, 1), lambda b: (0, 0)),
        ],
        out_shape=[
            jax.ShapeDtypeStruct((B, D, S), jnp.float32),
            jax.ShapeDtypeStruct((B, NQ, S), jnp.int32),
            jax.ShapeDtypeStruct((1, 1), jnp.float32),
            jax.ShapeDtypeStruct((1, 1), jnp.float32),
        ],
        scratch_shapes=[
            pltpu.VMEM((NQ, K), jnp.float32),
            pltpu.VMEM((1, 1), jnp.float32),
            pltpu.VMEM((NQ, K), jnp.float32),
            pltpu.VMEM((NQ, K, D), jnp.bfloat16),
            pltpu.VMEM((NQ, K, D), jnp.bfloat16),
            pltpu.VMEM((NQ, K, D), jnp.bfloat16),
        ],
    )(z, codebooks)
    return (quant,
            jnp.transpose(idx, (0, 2, 1)),
            loss[0, 0],
            perp[0, 0])
